# baseline (speedup 1.0000x reference)
import functools

import jax
import jax.numpy as jnp
from jax import lax
from jax.experimental import pallas as pl
from jax.experimental.pallas import tpu as pltpu
from jax.experimental.pallas import tpu_sc as plsc

_N = 10000
_E = 160000
_D = 256
_R = 7
_L = 3
_H = 128
_NS = 16
_K = 128
_NH = 5000
_CPT = 40
_CAP = _CPT * _NS * _K
_NACC = 5008
_NPT = 312
_RW = 104


def _mm_body(h_ref, w_ref, b_ref, out_ref):
    r = pl.program_id(1)
    h0 = h_ref[:, 0, :]
    h1 = h_ref[:, 1, :]
    w = w_ref[0]
    acc = jnp.dot(h0, w[:_H, :], preferred_element_type=jnp.float32)
    acc += jnp.dot(h1, w[_H:, :], preferred_element_type=jnp.float32)
    acc += jnp.where(r == _R, 1.0, 0.0) * b_ref[0]
    out_ref[0] = acc


def _tc_matmul(h2, w_all, b2, bn=1000):
    ni = _N // bn
    return pl.pallas_call(
        _mm_body,
        grid=(ni, _R + 1),
        in_specs=[
            pl.BlockSpec((bn, 2, _H), lambda i, r: (i, 0, 0)),
            pl.BlockSpec((1, _D, _D), lambda i, r: (r, 0, 0)),
            pl.BlockSpec((1, _D), lambda i, r: (0, 0)),
        ],
        out_specs=pl.BlockSpec((1, bn, _D), lambda i, r: (r, i, 0)),
        out_shape=jax.ShapeDtypeStruct((_R + 1, _N, _D), jnp.float32),
    )(h2, w_all, b2)


def _sc_body(hw_hbm, gidx_hbm, dst_hbm, out_hbm,
             gidx_s, dst_s, buf, acc, semg):
    c = lax.axis_index("c")
    s = lax.axis_index("s")
    base = _NH * c

    i0 = s * _NPT
    pltpu.sync_copy(hw_hbm.at[pl.ds(_R * _N + base + i0, _NPT)],
                    acc.at[pl.ds(i0, _NPT)])

    @pl.when(s == _NS - 1)
    def _init_last():
        last = _NS * _NPT
        pltpu.sync_copy(hw_hbm.at[pl.ds(_R * _N + base + last, _NH - last)],
                        acc.at[pl.ds(last, _NH - last)])

    pltpu.sync_copy(gidx_hbm.at[c, pl.ds(_CPT * s, _CPT)], gidx_s)
    pltpu.sync_copy(dst_hbm.at[c, pl.ds(_CPT * s, _CPT)], dst_s)
    plsc.subcore_barrier()

    for j in range(_CPT):
        pltpu.async_copy(hw_hbm.at[gidx_s.at[j]], buf, semg).wait()
        pltpu.sync_copy(buf, acc.at[dst_s.at[j]], add=True)

    plsc.subcore_barrier()

    for q in range(_NPT // _RW):
        r0 = i0 + q * _RW
        pltpu.sync_copy(acc.at[pl.ds(r0, _RW)], buf.at[pl.ds(0, _RW)])

        def relu_row(i, carry):
            for ch in range(2):
                for k in range(_H // 16):
                    sl = pl.ds(k * 16, 16)
                    buf[i, ch, sl] = jnp.maximum(buf[i, ch, sl], 0.0)
            return carry

        lax.fori_loop(0, _RW, relu_row, 0)
        pltpu.sync_copy(buf.at[pl.ds(0, _RW)],
                        out_hbm.at[pl.ds(base + r0, _RW)])

    @pl.when(s == _NS - 1)
    def _write_last():
        last = _NS * _NPT
        nr = _NH - last
        pltpu.sync_copy(acc.at[pl.ds(last, nr)], buf.at[pl.ds(0, nr)])

        def relu_row_t(i, carry):
            for ch in range(2):
                for k in range(_H // 16):
                    sl = pl.ds(k * 16, 16)
                    buf[i, ch, sl] = jnp.maximum(buf[i, ch, sl], 0.0)
            return carry

        lax.fori_loop(0, nr, relu_row_t, 0)
        pltpu.sync_copy(buf.at[pl.ds(0, nr)],
                        out_hbm.at[pl.ds(base + last, nr)])


_sc_edge = functools.partial(
    pl.kernel,
    _sc_body,
    out_type=jax.ShapeDtypeStruct((_N, 2, _H), jnp.float32),
    mesh=plsc.VectorSubcoreMesh(core_axis_name="c", subcore_axis_name="s"),
    scratch_types=[
        pltpu.VMEM((_CPT, _K), jnp.int32),
        pltpu.VMEM((_CPT, _K), jnp.int32),
        pltpu.VMEM((_K, 2, _H), jnp.float32),
        pltpu.VMEM_SHARED((_NACC, 2, _H), jnp.float32),
        pltpu.SemaphoreType.DMA,
    ],
)()


def kernel(x, edge_index, edge_type, node_position, W_rel, W_self, b):
    src = edge_index[0]
    dst = edge_index[1]

    gidx0 = edge_type * _N + src
    m = (dst >= _NH).astype(jnp.int32)
    c0 = jnp.cumsum(1 - m)
    c1 = jnp.cumsum(m)
    pos = jnp.where(m == 0, c0 - 1, _CAP + c1 - 1)
    gidxp = jnp.zeros((2 * _CAP,), jnp.int32).at[pos].set(gidx0, mode="drop")
    dstp = jnp.full((2 * _CAP,), _NH, jnp.int32).at[pos].set(
        dst - _NH * m, mode="drop")
    gidx3 = gidxp.reshape(2, _CAP // _K, _K)
    dst3 = dstp.reshape(2, _CAP // _K, _K)

    w_all = jnp.concatenate([W_rel, W_self[:, None]], axis=1)

    h2 = x.reshape(_N, 2, _H)
    outs = []
    for l in range(_L):
        hw = _tc_matmul(h2, w_all[l], b[l].reshape(1, _D))
        hw_flat = hw.reshape((_R + 1) * _N, 2, _H)
        h2 = _sc_edge(hw_flat, gidx3, dst3)
        outs.append(h2)

    node_feature = jnp.concatenate(
        [o.reshape(_N, _D) for o in outs], axis=-1)
    return node_feature, node_position
